# Initial kernel scaffold; baseline (speedup 1.0000x reference)
import jax
import jax.numpy as jnp
from jax.experimental import pallas as pl


def kernel(from_xy, to_xy, pos_embed, W, b):
    raise NotImplementedError("write your pallas kernel here")


# speedup vs baseline: 1.3745x; 1.3745x over previous
import functools

import jax
import jax.numpy as jnp
from jax import lax
from jax.experimental import pallas as pl
from jax.experimental.pallas import tpu as pltpu, tpu_sc as plsc

BOARD = 19
NPOS = BOARD * BOARD
D = 128
M = 16384
NPAD = 384
NC, NS, L = 2, 16, 16
NW = NC * NS
ROWS_PER_W = M // NW
CHUNK = 256
N_CHUNKS = ROWS_PER_W // CHUNK


def _table_body(p_ref, wt_ref, b_ref, o_ref):
    i = pl.program_id(0)
    acc = jnp.dot(p_ref[...], wt_ref[...], preferred_element_type=jnp.float32)
    bias = jnp.where(i == 0, b_ref[...], jnp.zeros_like(b_ref[...]))
    o_ref[...] = acc + bias


def _build_table(p_pad, wt, b_row):
    return pl.pallas_call(
        _table_body,
        grid=(2,),
        in_specs=[
            pl.BlockSpec((NPAD, D), lambda i: (0, 0)),
            pl.BlockSpec((D, D), lambda i: (i, 0)),
            pl.BlockSpec((1, D), lambda i: (0, 0)),
        ],
        out_specs=pl.BlockSpec((NPAD, D), lambda i: (i, 0)),
        out_shape=jax.ShapeDtypeStruct((2 * NPAD, D), jnp.float32),
    )(p_pad, wt, b_row)


def _sc_body(fxy_hbm, txy_hbm, tbl_hbm, out_hbm,
             fxy_v, txy_v, fi_v, ti_v, rows_a, rows_b, sem_a, sem_b):
    wid = lax.axis_index("s") * NC + lax.axis_index("c")
    lane2 = lax.iota(jnp.int32, L) * 2

    for c in range(N_CHUNKS):
        base = wid * ROWS_PER_W + c * CHUNK
        pltpu.sync_copy(fxy_hbm.at[pl.ds(2 * base, 2 * CHUNK)], fxy_v)
        pltpu.sync_copy(txy_hbm.at[pl.ds(2 * base, 2 * CHUNK)], txy_v)

        def idx_body(j, _):
            pos = j * (2 * L) + lane2
            fx = plsc.load_gather(fxy_v, [pos])
            fy = plsc.load_gather(fxy_v, [pos + 1])
            tx = plsc.load_gather(txy_v, [pos])
            ty = plsc.load_gather(txy_v, [pos + 1])
            fi = jnp.clip(fx * BOARD + fy, 0, NPOS - 1)
            ti = jnp.clip(tx * BOARD + ty, 0, NPOS - 1) + NPAD
            fi_v[pl.ds(j * L, L)] = fi
            ti_v[pl.ds(j * L, L)] = ti
            return 0

        lax.fori_loop(0, CHUNK // L, idx_body, 0, unroll=4)

        cp_a = pltpu.async_copy(tbl_hbm.at[fi_v], rows_a, sem_a)
        cp_b = pltpu.async_copy(tbl_hbm.at[ti_v], rows_b, sem_b)
        cp_a.wait()
        cp_b.wait()

        def relu_body(r, _):
            for k in range(D // L):
                a = rows_a[r, pl.ds(k * L, L)]
                bb = rows_b[r, pl.ds(k * L, L)]
                rows_a[r, pl.ds(k * L, L)] = jnp.maximum(a + bb, 0.0)
            return 0

        lax.fori_loop(0, CHUNK, relu_body, 0, unroll=2)

        pltpu.sync_copy(rows_a, out_hbm.at[pl.ds(base, CHUNK)])


_sc_lookup = functools.partial(
    pl.kernel,
    out_type=jax.ShapeDtypeStruct((M, D), jnp.float32),
    mesh=plsc.VectorSubcoreMesh(
        core_axis_name="c", subcore_axis_name="s", num_cores=NC, num_subcores=NS
    ),
    compiler_params=pltpu.CompilerParams(needs_layout_passes=False),
    scratch_types=[
        pltpu.VMEM((2 * CHUNK,), jnp.int32),
        pltpu.VMEM((2 * CHUNK,), jnp.int32),
        pltpu.VMEM((CHUNK,), jnp.int32),
        pltpu.VMEM((CHUNK,), jnp.int32),
        pltpu.VMEM((CHUNK, D), jnp.float32),
        pltpu.VMEM((CHUNK, D), jnp.float32),
        pltpu.SemaphoreType.DMA,
        pltpu.SemaphoreType.DMA,
    ],
)(_sc_body)


def kernel(from_xy, to_xy, pos_embed, W, b):
    p_pad = jnp.pad(pos_embed, ((0, NPAD - NPOS), (0, 0)))
    wt = W.T
    b_row = b.reshape(1, D)
    table = _build_table(p_pad, wt, b_row)
    fxy = from_xy.astype(jnp.int32).reshape(-1)
    txy = to_xy.astype(jnp.int32).reshape(-1)
    return _sc_lookup(fxy, txy, table)
